# baseline (speedup 1.0000x reference)
import numpy as np
import jax
import jax.numpy as jnp
from jax import lax
from jax.experimental import pallas as pl
from jax.experimental.pallas import tpu as pltpu, tpu_sc as plsc

MAXLEN = 200
DIM = 64
BATCH = 4096
SEQ = 200

NC = 2
NS = 16
NW = NC * NS
LANES = 16

B_PER_W = BATCH // NW
DT = DIM // 8
LINE = 8 * B_PER_W
TPP = LANES + 1


def _sinusoidal_pos_emb(maxlen, d_model):
    position = np.arange(maxlen)[:, np.newaxis]
    i = np.arange(d_model)[np.newaxis, :]
    angles = 1.0 / np.power(10000, 2 * (i // 2) / np.float32(d_model))
    angle_rads = position * angles
    angle_rads[:, 0::2] = np.sin(angle_rads[:, 0::2])
    angle_rads[:, 1::2] = np.cos(angle_rads[:, 1::2])
    return angle_rads.astype(np.float32)


_POS_NP = _sinusoidal_pos_emb(MAXLEN, DIM)


def _sc_body(xflat_hbm, pos_hbm, table_hbm, out_hbm,
             xb_v, xt_v, pos_v, rows0, rows1, blk0, blk1,
             tp,
             gsem0, gsem1, wsem0, wsem1):
    rows = (rows0, rows1)
    blk = (blk0, blk1)
    gsem = (gsem0, gsem1)
    wsem = (wsem0, wsem1)

    w = lax.axis_index("s") * NC + lax.axis_index("c")
    b0 = w * B_PER_W

    pltpu.sync_copy(xflat_hbm.at[pl.ds(b0 * SEQ, B_PER_W * SEQ)], xb_v)
    pltpu.sync_copy(pos_hbm, pos_v)

    iota = lax.iota(jnp.int32, LANES)
    cjs = [jnp.full((LANES,), j, jnp.int32) for j in range(LANES)]

    def xt_body(s, _):
        for bg in range(B_PER_W // LANES):
            idxv = (iota + bg * LANES) * SEQ + s
            xt_v[s, pl.ds(bg * LANES, LANES)] = plsc.load_gather(xb_v, [idxv])
        return 0

    lax.fori_loop(0, SEQ, xt_body, 0)

    pltpu.async_copy(table_hbm.at[xt_v.at[0]], rows[0], gsem[0])

    def pair_body(g, _):
        for par in range(2):
            s = 2 * g + par

            pltpu.make_async_copy(
                table_hbm.at[xt_v.at[s]], rows[par], gsem[par]).wait()

            @pl.when(s + 1 < SEQ)
            def _():
                pltpu.async_copy(table_hbm.at[xt_v.at[s + 1]],
                                 rows[1 - par], gsem[1 - par])

            @pl.when(s >= 2)
            def _():
                pltpu.make_async_copy(
                    blk[par], out_hbm.at[s - 2, :, w, :], wsem[par]).wait()

            pvecs = [pos_v[s, pl.ds(dg * LANES, LANES)]
                     for dg in range(DIM // LANES)]

            @plsc.parallel_loop(0, B_PER_W // LANES, step=1, unroll=4)
            def _(bg):
                for dg in range(DIM // LANES):
                    t = tp.at[par, bg, dg]
                    for i in range(LANES):
                        t[i, pl.ds(0, LANES)] = (
                            rows[par][bg * LANES + i,
                                      pl.ds(dg * LANES, LANES)]
                            + pvecs[dg])
                    for j in range(LANES):
                        col = plsc.load_gather(t, [iota, cjs[j]])
                        d = dg * LANES + j
                        blk[par][d // 8,
                                 pl.ds((d % 8) * B_PER_W + bg * LANES,
                                       LANES)] = col

            pltpu.async_copy(blk[par], out_hbm.at[s, :, w, :], wsem[par])
        return 0

    lax.fori_loop(0, SEQ // 2, pair_body, 0)

    pltpu.make_async_copy(blk[0], out_hbm.at[SEQ - 2, :, w, :],
                          wsem[0]).wait()
    pltpu.make_async_copy(blk[1], out_hbm.at[SEQ - 1, :, w, :],
                          wsem[1]).wait()


@jax.jit
def _embed(x_flat, token_table):
    pos = jnp.asarray(_POS_NP)
    mesh = plsc.VectorSubcoreMesh(core_axis_name="c", subcore_axis_name="s")
    fn = pl.kernel(
        _sc_body,
        out_type=jax.ShapeDtypeStruct((SEQ, DT, NW, LINE), jnp.float32),
        mesh=mesh,
        scratch_types=[
            pltpu.VMEM((B_PER_W * SEQ,), jnp.int32),
            pltpu.VMEM((SEQ, B_PER_W), jnp.int32),
            pltpu.VMEM((SEQ, DIM), jnp.float32),
            pltpu.VMEM((B_PER_W, DIM), jnp.float32),
            pltpu.VMEM((B_PER_W, DIM), jnp.float32),
            pltpu.VMEM((DT, LINE), jnp.float32),
            pltpu.VMEM((DT, LINE), jnp.float32),
            pltpu.VMEM((2, B_PER_W // LANES, DIM // LANES, LANES, TPP),
                       jnp.float32),
            pltpu.SemaphoreType.DMA,
            pltpu.SemaphoreType.DMA,
            pltpu.SemaphoreType.DMA,
            pltpu.SemaphoreType.DMA,
        ],
        compiler_params=pltpu.CompilerParams(use_tc_tiling_on_sc=False,
                                             needs_layout_passes=False),
    )
    return fn(x_flat, pos, token_table)


def kernel(x, token_table):
    x_flat = x.reshape(-1).astype(jnp.int32)
    out4 = _embed(x_flat, token_table)
    out = out4.reshape(SEQ, DT, NW, 8, B_PER_W)
    out = out.transpose(2, 4, 0, 1, 3)
    return out.reshape(BATCH, SEQ, DIM)


# speedup vs baseline: 1.1948x; 1.1948x over previous
import numpy as np
import jax
import jax.numpy as jnp
from jax import lax
from jax.experimental import pallas as pl
from jax.experimental.pallas import tpu as pltpu, tpu_sc as plsc

MAXLEN = 200
DIM = 64
BATCH = 4096
SEQ = 200

NC = 2
NS = 16
NW = NC * NS
LANES = 16

B_PER_W = BATCH // NW
DT = DIM // 8
LINE = 8 * B_PER_W
TPP = LANES + 1


def _sinusoidal_pos_emb(maxlen, d_model):
    position = np.arange(maxlen)[:, np.newaxis]
    i = np.arange(d_model)[np.newaxis, :]
    angles = 1.0 / np.power(10000, 2 * (i // 2) / np.float32(d_model))
    angle_rads = position * angles
    angle_rads[:, 0::2] = np.sin(angle_rads[:, 0::2])
    angle_rads[:, 1::2] = np.cos(angle_rads[:, 1::2])
    return angle_rads.astype(np.float32)


_POS_NP = _sinusoidal_pos_emb(MAXLEN, DIM)


def _sc_body(xflat_hbm, pos_hbm, table_hbm, out_hbm,
             xb_v, xt_v, pos_v, rows0, rows1, blk0, blk1,
             tp,
             gsem0, gsem1, wsem0, wsem1):
    rows = (rows0, rows1)
    blk = (blk0, blk1)
    gsem = (gsem0, gsem1)
    wsem = (wsem0, wsem1)

    w = lax.axis_index("s") * NC + lax.axis_index("c")
    b0 = w * B_PER_W

    pltpu.sync_copy(xflat_hbm.at[pl.ds(b0 * SEQ, B_PER_W * SEQ)], xb_v)
    pltpu.sync_copy(pos_hbm, pos_v)

    iota = lax.iota(jnp.int32, LANES)
    cjs = [jnp.full((LANES,), j, jnp.int32) for j in range(LANES)]

    def xt_body(s, _):
        for bg in range(B_PER_W // LANES):
            idxv = (iota + bg * LANES) * SEQ + s
            xt_v[s, pl.ds(bg * LANES, LANES)] = plsc.load_gather(xb_v, [idxv])
        return 0

    lax.fori_loop(0, SEQ, xt_body, 0)

    pltpu.async_copy(table_hbm.at[xt_v.at[0]], rows[0], gsem[0])

    def pair_body(g, _):
        for par in range(2):
            s = 2 * g + par

            pltpu.make_async_copy(
                table_hbm.at[xt_v.at[s]], rows[par], gsem[par]).wait()

            @pl.when(s + 1 < SEQ)
            def _():
                pltpu.async_copy(table_hbm.at[xt_v.at[s + 1]],
                                 rows[1 - par], gsem[1 - par])

            @pl.when(s >= 2)
            def _():
                pltpu.make_async_copy(
                    blk[par], out_hbm.at[s - 2, :, w, :], wsem[par]).wait()

            pvecs = [pos_v[s, pl.ds(dg * LANES, LANES)]
                     for dg in range(DIM // LANES)]

            @plsc.parallel_loop(0, B_PER_W // LANES, step=1, unroll=2)
            def _(bg):
                for dg in range(DIM // LANES):
                    t = tp.at[par, bg, dg]
                    for i in range(LANES):
                        t[i, pl.ds(0, LANES)] = (
                            rows[par][bg * LANES + i,
                                      pl.ds(dg * LANES, LANES)]
                            + pvecs[dg])
                    for j in range(LANES):
                        col = plsc.load_gather(t, [iota, cjs[j]])
                        d = dg * LANES + j
                        blk[par][d // 8,
                                 pl.ds((d % 8) * B_PER_W + bg * LANES,
                                       LANES)] = col

            pltpu.async_copy(blk[par], out_hbm.at[s, :, w, :], wsem[par])
        return 0

    lax.fori_loop(0, SEQ // 2, pair_body, 0)

    pltpu.make_async_copy(blk[0], out_hbm.at[SEQ - 2, :, w, :],
                          wsem[0]).wait()
    pltpu.make_async_copy(blk[1], out_hbm.at[SEQ - 1, :, w, :],
                          wsem[1]).wait()


@jax.jit
def _embed(x_flat, token_table):
    pos = jnp.asarray(_POS_NP)
    mesh = plsc.VectorSubcoreMesh(core_axis_name="c", subcore_axis_name="s")
    fn = pl.kernel(
        _sc_body,
        out_type=jax.ShapeDtypeStruct((SEQ, DT, NW, LINE), jnp.float32),
        mesh=mesh,
        scratch_types=[
            pltpu.VMEM((B_PER_W * SEQ,), jnp.int32),
            pltpu.VMEM((SEQ, B_PER_W), jnp.int32),
            pltpu.VMEM((SEQ, DIM), jnp.float32),
            pltpu.VMEM((B_PER_W, DIM), jnp.float32),
            pltpu.VMEM((B_PER_W, DIM), jnp.float32),
            pltpu.VMEM((DT, LINE), jnp.float32),
            pltpu.VMEM((DT, LINE), jnp.float32),
            pltpu.VMEM((2, B_PER_W // LANES, DIM // LANES, LANES, TPP),
                       jnp.float32),
            pltpu.SemaphoreType.DMA,
            pltpu.SemaphoreType.DMA,
            pltpu.SemaphoreType.DMA,
            pltpu.SemaphoreType.DMA,
        ],
        compiler_params=pltpu.CompilerParams(use_tc_tiling_on_sc=False,
                                             needs_layout_passes=False),
    )
    return fn(x_flat, pos, token_table)


def kernel(x, token_table):
    x_flat = x.reshape(-1).astype(jnp.int32)
    out4 = _embed(x_flat, token_table)
    out = out4.reshape(SEQ, DT, NW, 8, B_PER_W)
    out = out.transpose(2, 4, 0, 1, 3)
    return out.reshape(BATCH, SEQ, DIM)


# speedup vs baseline: 1.6980x; 1.4212x over previous
import numpy as np
import jax
import jax.numpy as jnp
from jax import lax
from jax.experimental import pallas as pl
from jax.experimental.pallas import tpu as pltpu, tpu_sc as plsc

MAXLEN = 200
DIM = 64
BATCH = 4096
SEQ = 200

NC = 2
NS = 16
NW = NC * NS
LANES = 16

B_PER_W = BATCH // NW
DT = DIM // 8
LINE = 8 * B_PER_W
TPP = LANES + 1


def _sinusoidal_pos_emb(maxlen, d_model):
    position = np.arange(maxlen)[:, np.newaxis]
    i = np.arange(d_model)[np.newaxis, :]
    angles = 1.0 / np.power(10000, 2 * (i // 2) / np.float32(d_model))
    angle_rads = position * angles
    angle_rads[:, 0::2] = np.sin(angle_rads[:, 0::2])
    angle_rads[:, 1::2] = np.cos(angle_rads[:, 1::2])
    return angle_rads.astype(np.float32)


_POS_NP = _sinusoidal_pos_emb(MAXLEN, DIM)


def _sc_body(xflat_hbm, pos_hbm, table_hbm, out_hbm,
             xb_v, xt_v, pos_v, rows0, rows1, blk0, blk1,
             tp,
             gsem0, gsem1, wsem0, wsem1):
    rows = (rows0, rows1)
    blk = (blk0, blk1)
    gsem = (gsem0, gsem1)
    wsem = (wsem0, wsem1)

    w = lax.axis_index("s") * NC + lax.axis_index("c")
    b0 = w * B_PER_W

    pltpu.sync_copy(xflat_hbm.at[pl.ds(b0 * SEQ, B_PER_W * SEQ)], xb_v)
    pltpu.sync_copy(pos_hbm, pos_v)

    iota = lax.iota(jnp.int32, LANES)
    cjs = [jnp.full((LANES,), j, jnp.int32) for j in range(LANES)]

    def xt_body(s, _):
        for bg in range(B_PER_W // LANES):
            idxv = (iota + bg * LANES) * SEQ + s
            xt_v[s, pl.ds(bg * LANES, LANES)] = plsc.load_gather(xb_v, [idxv])
        return 0

    lax.fori_loop(0, SEQ, xt_body, 0)

    pltpu.async_copy(table_hbm.at[xt_v.at[0]], rows[0], gsem[0])

    def pair_body(g, _):
        for par in range(2):
            s = 2 * g + par

            pltpu.make_async_copy(
                table_hbm.at[xt_v.at[s]], rows[par], gsem[par]).wait()

            @pl.when(s + 1 < SEQ)
            def _():
                pltpu.async_copy(table_hbm.at[xt_v.at[s + 1]],
                                 rows[1 - par], gsem[1 - par])

            @pl.when(s >= 2)
            def _():
                pltpu.make_async_copy(
                    blk[par], out_hbm.at[s - 2, :, w, :], wsem[par]).wait()

            NDG = DIM // LANES

            @plsc.parallel_loop(0, (B_PER_W // LANES) * NDG, step=1,
                                unroll=2)
            def _(ib):
                bg = ib // NDG
                dg = lax.rem(ib, NDG)
                pvec = pos_v[s, pl.ds(dg * LANES, LANES)]
                t = tp.at[par, ib]
                for i in range(LANES):
                    t[i, pl.ds(0, LANES)] = (
                        rows[par][bg * LANES + i,
                                  pl.ds(dg * LANES, LANES)]
                        + pvec)
                for j in range(LANES):
                    col = plsc.load_gather(t, [iota, cjs[j]])
                    blk[par][dg * 2 + j // 8,
                             pl.ds((j % 8) * B_PER_W + bg * LANES,
                                   LANES)] = col

            pltpu.async_copy(blk[par], out_hbm.at[s, :, w, :], wsem[par])
        return 0

    lax.fori_loop(0, SEQ // 2, pair_body, 0)

    pltpu.make_async_copy(blk[0], out_hbm.at[SEQ - 2, :, w, :],
                          wsem[0]).wait()
    pltpu.make_async_copy(blk[1], out_hbm.at[SEQ - 1, :, w, :],
                          wsem[1]).wait()


@jax.jit
def _embed(x_flat, token_table):
    pos = jnp.asarray(_POS_NP)
    mesh = plsc.VectorSubcoreMesh(core_axis_name="c", subcore_axis_name="s")
    fn = pl.kernel(
        _sc_body,
        out_type=jax.ShapeDtypeStruct((SEQ, DT, NW, LINE), jnp.float32),
        mesh=mesh,
        scratch_types=[
            pltpu.VMEM((B_PER_W * SEQ,), jnp.int32),
            pltpu.VMEM((SEQ, B_PER_W), jnp.int32),
            pltpu.VMEM((SEQ, DIM), jnp.float32),
            pltpu.VMEM((B_PER_W, DIM), jnp.float32),
            pltpu.VMEM((B_PER_W, DIM), jnp.float32),
            pltpu.VMEM((DT, LINE), jnp.float32),
            pltpu.VMEM((DT, LINE), jnp.float32),
            pltpu.VMEM((2, (B_PER_W // LANES) * (DIM // LANES), LANES, TPP),
                       jnp.float32),
            pltpu.SemaphoreType.DMA,
            pltpu.SemaphoreType.DMA,
            pltpu.SemaphoreType.DMA,
            pltpu.SemaphoreType.DMA,
        ],
        compiler_params=pltpu.CompilerParams(use_tc_tiling_on_sc=False,
                                             needs_layout_passes=False),
    )
    return fn(x_flat, pos, token_table)


def kernel(x, token_table):
    x_flat = x.reshape(-1).astype(jnp.int32)
    out4 = _embed(x_flat, token_table)
    out = out4.reshape(SEQ, DT, NW, 8, B_PER_W)
    out = out.transpose(2, 4, 0, 1, 3)
    return out.reshape(BATCH, SEQ, DIM)


# speedup vs baseline: 2.2619x; 1.3321x over previous
import numpy as np
import jax
import jax.numpy as jnp
from jax import lax
from jax.experimental import pallas as pl
from jax.experimental.pallas import tpu as pltpu, tpu_sc as plsc

MAXLEN = 200
DIM = 64
BATCH = 4096
SEQ = 200

NC = 2
NS = 16
NW = NC * NS
LANES = 16

B_PER_W = BATCH // NW
DT = DIM // 8
LINE = 8 * B_PER_W
TPP = LANES + 1


def _sinusoidal_pos_emb(maxlen, d_model):
    position = np.arange(maxlen)[:, np.newaxis]
    i = np.arange(d_model)[np.newaxis, :]
    angles = 1.0 / np.power(10000, 2 * (i // 2) / np.float32(d_model))
    angle_rads = position * angles
    angle_rads[:, 0::2] = np.sin(angle_rads[:, 0::2])
    angle_rads[:, 1::2] = np.cos(angle_rads[:, 1::2])
    return angle_rads.astype(np.float32)


_POS_NP = _sinusoidal_pos_emb(MAXLEN, DIM)


def _sc_body(xflat_hbm, pos_hbm, table_hbm, out_hbm,
             xb_v, xt_v, pos_v, rows0, rows1, blk0, blk1,
             tp,
             gsem0, gsem1, wsem0, wsem1):
    rows = (rows0, rows1)
    blk = (blk0, blk1)
    gsem = (gsem0, gsem1)
    wsem = (wsem0, wsem1)

    w = lax.axis_index("s") * NC + lax.axis_index("c")
    b0 = w * B_PER_W

    pltpu.sync_copy(xflat_hbm.at[pl.ds(b0 * SEQ, B_PER_W * SEQ)], xb_v)
    pltpu.sync_copy(pos_hbm, pos_v)

    iota = lax.iota(jnp.int32, LANES)
    cjs = [jnp.full((LANES,), j, jnp.int32) for j in range(LANES)]

    def xt_body(s, _):
        for bg in range(B_PER_W // LANES):
            idxv = (iota + bg * LANES) * SEQ + s
            xt_v[s, pl.ds(bg * LANES, LANES)] = plsc.load_gather(xb_v, [idxv])
        return 0

    lax.fori_loop(0, SEQ, xt_body, 0)

    pltpu.async_copy(table_hbm.at[xt_v.at[0]], rows[0], gsem[0])

    def pair_body(g, _):
        for par in range(2):
            s = 2 * g + par

            pltpu.make_async_copy(
                table_hbm.at[xt_v.at[s]], rows[par], gsem[par]).wait()

            @pl.when(s + 1 < SEQ)
            def _():
                pltpu.async_copy(table_hbm.at[xt_v.at[s + 1]],
                                 rows[1 - par], gsem[1 - par])

            @pl.when(s >= 2)
            def _():
                pltpu.make_async_copy(
                    blk[par], out_hbm.at[s - 2, :, w, :], wsem[par]).wait()

            NDG = DIM // LANES

            @plsc.parallel_loop(0, (B_PER_W // LANES) * NDG, step=1,
                                unroll=4)
            def _(ib):
                bg = ib // NDG
                dg = lax.rem(ib, NDG)
                pvec = pos_v[s, pl.ds(dg * LANES, LANES)]
                t = tp.at[par, ib]
                for i in range(LANES):
                    t[i, pl.ds(0, LANES)] = (
                        rows[par][bg * LANES + i,
                                  pl.ds(dg * LANES, LANES)]
                        + pvec)
                for j in range(LANES):
                    col = plsc.load_gather(t, [iota, cjs[j]])
                    blk[par][dg * 2 + j // 8,
                             pl.ds((j % 8) * B_PER_W + bg * LANES,
                                   LANES)] = col

            pltpu.async_copy(blk[par], out_hbm.at[s, :, w, :], wsem[par])
        return 0

    lax.fori_loop(0, SEQ // 2, pair_body, 0)

    pltpu.make_async_copy(blk[0], out_hbm.at[SEQ - 2, :, w, :],
                          wsem[0]).wait()
    pltpu.make_async_copy(blk[1], out_hbm.at[SEQ - 1, :, w, :],
                          wsem[1]).wait()


@jax.jit
def _embed(x_flat, token_table):
    pos = jnp.asarray(_POS_NP)
    mesh = plsc.VectorSubcoreMesh(core_axis_name="c", subcore_axis_name="s")
    fn = pl.kernel(
        _sc_body,
        out_type=jax.ShapeDtypeStruct((SEQ, DT, NW, LINE), jnp.float32),
        mesh=mesh,
        scratch_types=[
            pltpu.VMEM((B_PER_W * SEQ,), jnp.int32),
            pltpu.VMEM((SEQ, B_PER_W), jnp.int32),
            pltpu.VMEM((SEQ, DIM), jnp.float32),
            pltpu.VMEM((B_PER_W, DIM), jnp.float32),
            pltpu.VMEM((B_PER_W, DIM), jnp.float32),
            pltpu.VMEM((DT, LINE), jnp.float32),
            pltpu.VMEM((DT, LINE), jnp.float32),
            pltpu.VMEM((2, (B_PER_W // LANES) * (DIM // LANES), LANES, TPP),
                       jnp.float32),
            pltpu.SemaphoreType.DMA,
            pltpu.SemaphoreType.DMA,
            pltpu.SemaphoreType.DMA,
            pltpu.SemaphoreType.DMA,
        ],
        compiler_params=pltpu.CompilerParams(use_tc_tiling_on_sc=False,
                                             needs_layout_passes=False),
    )
    return fn(x_flat, pos, token_table)


def kernel(x, token_table):
    x_flat = x.reshape(-1).astype(jnp.int32)
    out4 = _embed(x_flat, token_table)
    out = out4.reshape(SEQ, DT, NW, 8, B_PER_W)
    out = out.transpose(2, 4, 0, 1, 3)
    return out.reshape(BATCH, SEQ, DIM)


# speedup vs baseline: 2.2973x; 1.0156x over previous
import numpy as np
import jax
import jax.numpy as jnp
from jax import lax
from jax.experimental import pallas as pl
from jax.experimental.pallas import tpu as pltpu, tpu_sc as plsc

MAXLEN = 200
DIM = 64
BATCH = 4096
SEQ = 200

NC = 2
NS = 16
NW = NC * NS
LANES = 16

B_PER_W = BATCH // NW
DT = DIM // 8
LINE = 8 * B_PER_W
TPP = LANES + 1


def _sinusoidal_pos_emb(maxlen, d_model):
    position = np.arange(maxlen)[:, np.newaxis]
    i = np.arange(d_model)[np.newaxis, :]
    angles = 1.0 / np.power(10000, 2 * (i // 2) / np.float32(d_model))
    angle_rads = position * angles
    angle_rads[:, 0::2] = np.sin(angle_rads[:, 0::2])
    angle_rads[:, 1::2] = np.cos(angle_rads[:, 1::2])
    return angle_rads.astype(np.float32)


_POS_NP = _sinusoidal_pos_emb(MAXLEN, DIM)


def _sc_body(xflat_hbm, pos_hbm, table_hbm, out_hbm,
             xb_v, xt_v, pos_v, rows0, rows1, blk0, blk1,
             tp,
             gsem0, gsem1, wsem0, wsem1):
    rows = (rows0, rows1)
    blk = (blk0, blk1)
    gsem = (gsem0, gsem1)
    wsem = (wsem0, wsem1)

    w = lax.axis_index("s") * NC + lax.axis_index("c")
    b0 = w * B_PER_W

    pltpu.sync_copy(xflat_hbm.at[pl.ds(b0 * SEQ, B_PER_W * SEQ)], xb_v)
    pltpu.sync_copy(pos_hbm, pos_v)

    iota = lax.iota(jnp.int32, LANES)
    cjs = [jnp.full((LANES,), j, jnp.int32) for j in range(LANES)]

    def xt_body(s, _):
        for bg in range(B_PER_W // LANES):
            idxv = (iota + bg * LANES) * SEQ + s
            xt_v[s, pl.ds(bg * LANES, LANES)] = plsc.load_gather(xb_v, [idxv])
        return 0

    lax.fori_loop(0, SEQ, xt_body, 0)

    pltpu.async_copy(table_hbm.at[xt_v.at[0]], rows[0], gsem[0])

    def pair_body(g, _):
        for par in range(2):
            s = 2 * g + par

            pltpu.make_async_copy(
                table_hbm.at[xt_v.at[s]], rows[par], gsem[par]).wait()

            @pl.when(s + 1 < SEQ)
            def _():
                pltpu.async_copy(table_hbm.at[xt_v.at[s + 1]],
                                 rows[1 - par], gsem[1 - par])

            @pl.when(s >= 2)
            def _():
                pltpu.make_async_copy(
                    blk[par], out_hbm.at[s - 2, :, w, :], wsem[par]).wait()

            NDG = DIM // LANES

            @plsc.parallel_loop(0, (B_PER_W // LANES) * NDG, step=1,
                                unroll=8)
            def _(ib):
                bg = ib // NDG
                dg = lax.rem(ib, NDG)
                pvec = pos_v[s, pl.ds(dg * LANES, LANES)]
                t = tp.at[par, ib]
                for i in range(LANES):
                    t[i, pl.ds(0, LANES)] = (
                        rows[par][bg * LANES + i,
                                  pl.ds(dg * LANES, LANES)]
                        + pvec)
                for j in range(LANES):
                    col = plsc.load_gather(t, [iota, cjs[j]])
                    blk[par][dg * 2 + j // 8,
                             pl.ds((j % 8) * B_PER_W + bg * LANES,
                                   LANES)] = col

            pltpu.async_copy(blk[par], out_hbm.at[s, :, w, :], wsem[par])
        return 0

    lax.fori_loop(0, SEQ // 2, pair_body, 0)

    pltpu.make_async_copy(blk[0], out_hbm.at[SEQ - 2, :, w, :],
                          wsem[0]).wait()
    pltpu.make_async_copy(blk[1], out_hbm.at[SEQ - 1, :, w, :],
                          wsem[1]).wait()


@jax.jit
def _embed(x_flat, token_table):
    pos = jnp.asarray(_POS_NP)
    mesh = plsc.VectorSubcoreMesh(core_axis_name="c", subcore_axis_name="s")
    fn = pl.kernel(
        _sc_body,
        out_type=jax.ShapeDtypeStruct((SEQ, DT, NW, LINE), jnp.float32),
        mesh=mesh,
        scratch_types=[
            pltpu.VMEM((B_PER_W * SEQ,), jnp.int32),
            pltpu.VMEM((SEQ, B_PER_W), jnp.int32),
            pltpu.VMEM((SEQ, DIM), jnp.float32),
            pltpu.VMEM((B_PER_W, DIM), jnp.float32),
            pltpu.VMEM((B_PER_W, DIM), jnp.float32),
            pltpu.VMEM((DT, LINE), jnp.float32),
            pltpu.VMEM((DT, LINE), jnp.float32),
            pltpu.VMEM((2, (B_PER_W // LANES) * (DIM // LANES), LANES, TPP),
                       jnp.float32),
            pltpu.SemaphoreType.DMA,
            pltpu.SemaphoreType.DMA,
            pltpu.SemaphoreType.DMA,
            pltpu.SemaphoreType.DMA,
        ],
        compiler_params=pltpu.CompilerParams(use_tc_tiling_on_sc=False,
                                             needs_layout_passes=False),
    )
    return fn(x_flat, pos, token_table)


def kernel(x, token_table):
    x_flat = x.reshape(-1).astype(jnp.int32)
    out4 = _embed(x_flat, token_table)
    out = out4.reshape(SEQ, DT, NW, 8, B_PER_W)
    out = out.transpose(2, 4, 0, 1, 3)
    return out.reshape(BATCH, SEQ, DIM)


# speedup vs baseline: 2.3197x; 1.0098x over previous
import numpy as np
import jax
import jax.numpy as jnp
from jax import lax
from jax.experimental import pallas as pl
from jax.experimental.pallas import tpu as pltpu, tpu_sc as plsc

MAXLEN = 200
DIM = 64
BATCH = 4096
SEQ = 200

NC = 2
NS = 16
NW = NC * NS
LANES = 16

B_PER_W = BATCH // NW
DT = DIM // 8
LINE = 8 * B_PER_W
TPP = LANES + 1


def _sinusoidal_pos_emb(maxlen, d_model):
    position = np.arange(maxlen)[:, np.newaxis]
    i = np.arange(d_model)[np.newaxis, :]
    angles = 1.0 / np.power(10000, 2 * (i // 2) / np.float32(d_model))
    angle_rads = position * angles
    angle_rads[:, 0::2] = np.sin(angle_rads[:, 0::2])
    angle_rads[:, 1::2] = np.cos(angle_rads[:, 1::2])
    return angle_rads.astype(np.float32)


_POS_NP = _sinusoidal_pos_emb(MAXLEN, DIM)


def _sc_body(xflat_hbm, pos_hbm, table_hbm, out_hbm,
             xb_v, xt_v, pos_v, rows0, rows1, blk0, blk1,
             tp,
             gsem0, gsem1, wsem0, wsem1):
    rows = (rows0, rows1)
    blk = (blk0, blk1)
    gsem = (gsem0, gsem1)
    wsem = (wsem0, wsem1)

    w = lax.axis_index("s") * NC + lax.axis_index("c")
    b0 = w * B_PER_W

    pltpu.sync_copy(xflat_hbm.at[pl.ds(b0 * SEQ, B_PER_W * SEQ)], xb_v)
    pltpu.sync_copy(pos_hbm, pos_v)

    iota = lax.iota(jnp.int32, LANES)
    fjs = [iota * TPP + j for j in range(LANES)]

    def xt_body(s, _):
        for bg in range(B_PER_W // LANES):
            idxv = (iota + bg * LANES) * SEQ + s
            xt_v[s, pl.ds(bg * LANES, LANES)] = plsc.load_gather(xb_v, [idxv])
        return 0

    lax.fori_loop(0, SEQ, xt_body, 0)

    pltpu.async_copy(table_hbm.at[xt_v.at[0]], rows[0], gsem[0])

    def pair_body(g, _):
        for par in range(2):
            s = 2 * g + par

            pltpu.make_async_copy(
                table_hbm.at[xt_v.at[s]], rows[par], gsem[par]).wait()

            @pl.when(s + 1 < SEQ)
            def _():
                pltpu.async_copy(table_hbm.at[xt_v.at[s + 1]],
                                 rows[1 - par], gsem[1 - par])

            @pl.when(s >= 2)
            def _():
                pltpu.make_async_copy(
                    blk[par], out_hbm.at[s - 2, :, w, :], wsem[par]).wait()

            NDG = DIM // LANES

            @plsc.parallel_loop(0, (B_PER_W // LANES) * NDG, step=1,
                                unroll=8)
            def _(ib):
                bg = ib // NDG
                dg = lax.rem(ib, NDG)
                pvec = pos_v[s, pl.ds(dg * LANES, LANES)]
                t = tp.at[par, ib]
                for i in range(LANES):
                    t[pl.ds(i * TPP, LANES)] = (
                        rows[par][bg * LANES + i,
                                  pl.ds(dg * LANES, LANES)]
                        + pvec)
                for j in range(LANES):
                    col = plsc.load_gather(t, [fjs[j]])
                    blk[par][dg * 2 + j // 8,
                             pl.ds((j % 8) * B_PER_W + bg * LANES,
                                   LANES)] = col

            pltpu.async_copy(blk[par], out_hbm.at[s, :, w, :], wsem[par])
        return 0

    lax.fori_loop(0, SEQ // 2, pair_body, 0)

    pltpu.make_async_copy(blk[0], out_hbm.at[SEQ - 2, :, w, :],
                          wsem[0]).wait()
    pltpu.make_async_copy(blk[1], out_hbm.at[SEQ - 1, :, w, :],
                          wsem[1]).wait()


@jax.jit
def _embed(x_flat, token_table):
    pos = jnp.asarray(_POS_NP)
    mesh = plsc.VectorSubcoreMesh(core_axis_name="c", subcore_axis_name="s")
    fn = pl.kernel(
        _sc_body,
        out_type=jax.ShapeDtypeStruct((SEQ, DT, NW, LINE), jnp.float32),
        mesh=mesh,
        scratch_types=[
            pltpu.VMEM((B_PER_W * SEQ,), jnp.int32),
            pltpu.VMEM((SEQ, B_PER_W), jnp.int32),
            pltpu.VMEM((SEQ, DIM), jnp.float32),
            pltpu.VMEM((B_PER_W, DIM), jnp.float32),
            pltpu.VMEM((B_PER_W, DIM), jnp.float32),
            pltpu.VMEM((DT, LINE), jnp.float32),
            pltpu.VMEM((DT, LINE), jnp.float32),
            pltpu.VMEM((2, (B_PER_W // LANES) * (DIM // LANES), LANES * TPP),
                       jnp.float32),
            pltpu.SemaphoreType.DMA,
            pltpu.SemaphoreType.DMA,
            pltpu.SemaphoreType.DMA,
            pltpu.SemaphoreType.DMA,
        ],
        compiler_params=pltpu.CompilerParams(use_tc_tiling_on_sc=False,
                                             needs_layout_passes=False),
    )
    return fn(x_flat, pos, token_table)


def kernel(x, token_table):
    x_flat = x.reshape(-1).astype(jnp.int32)
    out4 = _embed(x_flat, token_table)
    out = out4.reshape(SEQ, DT, NW, 8, B_PER_W)
    out = out.transpose(2, 4, 0, 1, 3)
    return out.reshape(BATCH, SEQ, DIM)
